# Initial kernel scaffold; baseline (speedup 1.0000x reference)
import jax
import jax.numpy as jnp
from jax.experimental import pallas as pl


def kernel(x, edge_index, edge_attr, mask_node_indices, prelu_a, W_enc, emb1, emb2, W1, b1, W2, b2):
    raise NotImplementedError("write your pallas kernel here")


# speedup vs baseline: 10.1678x; 10.1678x over previous
import jax
import jax.numpy as jnp
from jax import lax
from jax.experimental import pallas as pl
from jax.experimental.pallas import tpu as pltpu
from jax.experimental.pallas import tpu_sc as plsc

NC = 2
NS = 16
NW = NC * NS
CHUNK = 128
CPAD = 32


def _stage1_prelu_mask(x, mask_idx_2d, prelu_a_2d, blk):
    n, d = x.shape
    grid = n // blk

    def body(x_ref, m_ref, a_ref, o_ref):
        i = pl.program_id(0)
        xb = x_ref[...]
        a = a_ref[0, 0]
        pr = jnp.where(xb >= 0.0, xb, a * xb)
        ids = i * blk + lax.broadcasted_iota(jnp.int32, (blk, 1), 0)
        hit = jnp.any(ids == m_ref[...], axis=1, keepdims=True)
        o_ref[...] = jnp.where(hit, 0.0, pr)

    return pl.pallas_call(
        body,
        grid=(grid,),
        in_specs=[
            pl.BlockSpec((blk, d), lambda i: (i, 0)),
            pl.BlockSpec(mask_idx_2d.shape, lambda i: (0, 0)),
            pl.BlockSpec((1, 1), lambda i: (0, 0), memory_space=pltpu.SMEM),
        ],
        out_specs=pl.BlockSpec((blk, d), lambda i: (i, 0)),
        out_shape=jax.ShapeDtypeStruct((n, d), jnp.float32),
    )(x, mask_idx_2d, prelu_a_2d)


def _make_sc_kernel(n, d, e):
    nchunks = e // CHUNK
    per_w = -(-nchunks // NW)
    rows_t = (n // NS) // 8 * 8
    tail = n - rows_t * NS
    cnt_pad = -(-(n * CPAD) // (NS * 1024)) * (NS * 1024)
    cnt_t = cnt_pad // NS
    ZC = 2048
    n_full = rows_t // CHUNK
    rem = rows_t - n_full * CHUNK

    mesh = plsc.VectorSubcoreMesh(core_axis_name="c", subcore_axis_name="s")

    def body(p_hbm, ei_hbm, a0_hbm, a1_hbm, aggr_out, cnt_out,
             src_v, dst_v, a0_v, a1_v, fidx_v, rows_v, ones_v, zc_v, sem,
             aggr_sp, cnt_sp):
        cid = lax.axis_index("c")
        sid = lax.axis_index("s")
        wid = sid * NC + cid

        zero16 = jnp.zeros((16,), jnp.float32)
        one16 = jnp.ones((16,), jnp.float32)

        def zrow(r, _):
            for c in range(d // 16):
                rows_v[r, pl.ds(16 * c, 16)] = zero16
            return 0
        lax.fori_loop(0, CHUNK, zrow, 0)

        def zcnt(i, _):
            zc_v[pl.ds(i * 16, 16)] = zero16
            return 0
        lax.fori_loop(0, ZC // 16, zcnt, 0)

        for k in range(CHUNK // 16):
            ones_v[pl.ds(16 * k, 16)] = one16

        row0 = sid * rows_t
        for k in range(n_full):
            pltpu.sync_copy(rows_v, aggr_sp.at[pl.ds(row0 + k * CHUNK, CHUNK)])
        if rem:
            pltpu.sync_copy(rows_v.at[pl.ds(0, rem)],
                            aggr_sp.at[pl.ds(row0 + n_full * CHUNK, rem)])
        if tail:
            @pl.when(sid == NS - 1)
            def _():
                pltpu.sync_copy(rows_v.at[pl.ds(0, tail)],
                                aggr_sp.at[pl.ds(NS * rows_t, tail)])
        off0 = sid * cnt_t
        for k in range(cnt_t // ZC):
            pltpu.sync_copy(zc_v, cnt_sp.at[pl.ds(off0 + k * ZC, ZC)])

        plsc.subcore_barrier()

        def step(i, _):
            chunk = wid * per_w + i

            @pl.when(chunk < nchunks)
            def _():
                base = chunk * CHUNK
                pltpu.sync_copy(ei_hbm.at[0, pl.ds(base, CHUNK)], src_v)
                pltpu.sync_copy(ei_hbm.at[1, pl.ds(base, CHUNK)], dst_v)
                pltpu.sync_copy(a0_hbm.at[pl.ds(base, CHUNK)], a0_v)
                pltpu.sync_copy(a1_hbm.at[pl.ds(base, CHUNK)], a1_v)
                pltpu.async_copy(p_hbm.at[src_v], rows_v, sem).wait()
                pltpu.sync_copy(rows_v, aggr_sp.at[dst_v], add=True)
                for k in range(CHUNK // 16):
                    sl = pl.ds(16 * k, 16)
                    dd = dst_v[sl]
                    aa = jnp.minimum(jnp.maximum(a0_v[sl], 0), 5)
                    bb = jnp.minimum(jnp.maximum(a1_v[sl], 0), 2)
                    fidx_v[sl] = dd * CPAD + aa * 3 + bb
                pltpu.sync_copy(ones_v, cnt_sp.at[fidx_v], add=True)
            return 0

        lax.fori_loop(0, per_w, step, 0)

        plsc.subcore_barrier()

        pltpu.sync_copy(aggr_sp.at[pl.ds(row0, rows_t)],
                        aggr_out.at[cid, pl.ds(row0, rows_t)])
        if tail:
            @pl.when(sid == NS - 1)
            def _():
                pltpu.sync_copy(aggr_sp.at[pl.ds(NS * rows_t, tail)],
                                aggr_out.at[cid, pl.ds(NS * rows_t, tail)])
        pltpu.sync_copy(cnt_sp.at[pl.ds(off0, cnt_t)],
                        cnt_out.at[cid, pl.ds(off0, cnt_t)])

    return pl.kernel(
        body,
        out_type=[
            jax.ShapeDtypeStruct((NC, n, d), jnp.float32),
            jax.ShapeDtypeStruct((NC, cnt_pad), jnp.float32),
        ],
        mesh=mesh,
        scratch_types=[
            pltpu.VMEM((CHUNK,), jnp.int32),
            pltpu.VMEM((CHUNK,), jnp.int32),
            pltpu.VMEM((CHUNK,), jnp.int32),
            pltpu.VMEM((CHUNK,), jnp.int32),
            pltpu.VMEM((CHUNK,), jnp.int32),
            pltpu.VMEM((CHUNK, d), jnp.float32),
            pltpu.VMEM((CHUNK,), jnp.float32),
            pltpu.VMEM((ZC,), jnp.float32),
            pltpu.SemaphoreType.DMA,
            pltpu.VMEM_SHARED((n, d), jnp.float32),
            pltpu.VMEM_SHARED((cnt_pad,), jnp.float32),
        ],
    )


def _stage3_mlp(aggr, p, cnt3, W_enc, Tpad, W1, b1_2d, W2, b2_2d, blk):
    n, d = p.shape
    dh = W1.shape[0]
    grid = n // blk
    f32 = jnp.float32

    def body(g_ref, p_ref, c_ref, we_ref, t_ref, w1_ref, b1_ref, w2_ref,
             b2_ref, o_ref):
        g = g_ref[0] + g_ref[1] + p_ref[...]
        acc = lax.dot_general(g, we_ref[...], (((1,), (1,)), ((), ())),
                              preferred_element_type=f32)
        c = c_ref[0] + c_ref[1]
        acc = acc + jnp.dot(c, t_ref[...], preferred_element_type=f32)
        acc = acc + t_ref[12:13, :]
        h1 = lax.dot_general(acc, w1_ref[...], (((1,), (1,)), ((), ())),
                             preferred_element_type=f32) + b1_ref[...]
        h1 = jnp.maximum(h1, 0.0)
        o_ref[...] = lax.dot_general(h1, w2_ref[...], (((1,), (1,)), ((), ())),
                                     preferred_element_type=f32) + b2_ref[...]

    return pl.pallas_call(
        body,
        grid=(grid,),
        in_specs=[
            pl.BlockSpec((NC, blk, d), lambda i: (0, i, 0)),
            pl.BlockSpec((blk, d), lambda i: (i, 0)),
            pl.BlockSpec((NC, blk, CPAD), lambda i: (0, i, 0)),
            pl.BlockSpec((d, d), lambda i: (0, 0)),
            pl.BlockSpec((CPAD, d), lambda i: (0, 0)),
            pl.BlockSpec((dh, d), lambda i: (0, 0)),
            pl.BlockSpec((1, dh), lambda i: (0, 0)),
            pl.BlockSpec((d, dh), lambda i: (0, 0)),
            pl.BlockSpec((1, d), lambda i: (0, 0)),
        ],
        out_specs=pl.BlockSpec((blk, d), lambda i: (i, 0)),
        out_shape=jax.ShapeDtypeStruct((n, d), jnp.float32),
    )(aggr, p, cnt3, W_enc, Tpad, W1, b1_2d, W2, b2_2d)


def kernel(x, edge_index, edge_attr, mask_node_indices, prelu_a, W_enc,
           emb1, emb2, W1, b1, W2, b2):
    n, d = x.shape
    e = edge_index.shape[1]
    nm = mask_node_indices.shape[0]

    padw = -(-nm // 128) * 128
    m2d = jnp.concatenate(
        [mask_node_indices,
         jnp.full((padw - nm,), -1, mask_node_indices.dtype)]).reshape(1, padw)
    a2d = prelu_a.reshape(1, 1)
    a0 = edge_attr[:, 0]
    a1 = edge_attr[:, 1]
    T = (emb1[:, None, :] + emb2[None, :, :]).reshape(-1, d)
    Tpad = jnp.concatenate([T, jnp.zeros((CPAD - T.shape[0], d), T.dtype)])
    b1_2d = b1.reshape(1, -1)
    b2_2d = b2.reshape(1, -1)

    p = _stage1_prelu_mask(x, m2d, a2d, blk=1000)

    aggr, cnt = _make_sc_kernel(n, d, e)(p, edge_index, a0, a1)
    cnt3 = cnt[:, :n * CPAD].reshape(NC, n, CPAD)

    return _stage3_mlp(aggr, p, cnt3, W_enc, Tpad, W1, b1_2d, W2, b2_2d,
                       blk=1000)
